# baseline (speedup 1.0000x reference)
import functools
import jax
import jax.numpy as jnp
from jax import lax
from jax.experimental import pallas as pl
from jax.experimental.pallas import tpu as pltpu
from jax.experimental.pallas import tpu_sc as plsc

NC, NS = 2, 16
NW = NC * NS
N = 10000
NP = 10240
RPT = NP // NS
E = 320000
EP = 327680
EPT = EP // NW
CH = 128
NCHUNK = EPT // CH
ECH = 64
ENCH = EPT // ECH
D = 128
G = 128
DEGW = 128
BR = 512
NB = NP // BR

_mesh = plsc.VectorSubcoreMesh(core_axis_name="c", subcore_axis_name="s",
                               num_cores=NC, num_subcores=NS)


@functools.partial(
    pl.kernel,
    out_type=jax.ShapeDtypeStruct((NC, NP, DEGW), jnp.float32),
    mesh=_mesh,
    scratch_types=[
        pltpu.VMEM((NCHUNK, CH), jnp.int32),
        pltpu.VMEM((CH, DEGW), jnp.float32),
        pltpu.VMEM_SHARED((NP, DEGW), jnp.float32),
    ],
)
def _sc_degree(dst_hbm, ones_hbm, zeros_hbm, deg_out, idx_v, ones_v, deg_sh):
    cid = lax.axis_index("c")
    sid = lax.axis_index("s")
    wid = sid * NC + cid
    pltpu.sync_copy(zeros_hbm, deg_sh.at[pl.ds(sid * RPT, RPT)])
    pltpu.sync_copy(dst_hbm.at[wid], idx_v)
    pltpu.sync_copy(ones_hbm, ones_v)
    plsc.subcore_barrier()

    def chunk(c, carry):
        pltpu.sync_copy(ones_v, deg_sh.at[idx_v.at[c]], add=True)
        return carry

    lax.fori_loop(0, NCHUNK, chunk, 0)
    plsc.subcore_barrier()
    pltpu.sync_copy(deg_sh.at[pl.ds(sid * RPT, RPT)],
                    deg_out.at[cid, pl.ds(sid * RPT, RPT)])


RING = 4
IDXR = 8
UNROLL = 8


@functools.partial(
    pl.kernel,
    out_type=jax.ShapeDtypeStruct((NC, NP, D), jnp.float32),
    mesh=_mesh,
    scratch_types=[
        pltpu.VMEM((IDXR, 2, ECH), jnp.int32),
        pltpu.VMEM((RING, ECH, D), jnp.float32),
        pltpu.VMEM_SHARED((NP, D), jnp.float32),
        pltpu.SemaphoreType.DMA((IDXR,)),
        pltpu.SemaphoreType.DMA((RING,)),
    ],
)
def _sc_edge_agg(y_hbm, e_hbm, zeros_hbm, acc_out,
                 idxr_v, stage_v, acc_sh, isem, gsem):
    cid = lax.axis_index("c")
    sid = lax.axis_index("s")
    wid = sid * NC + cid
    pltpu.sync_copy(zeros_hbm, acc_sh.at[pl.ds(sid * RPT, RPT)])
    for u in range(IDXR):
        pltpu.async_copy(e_hbm.at[wid, u], idxr_v.at[u], isem.at[u])
    plsc.subcore_barrier()
    for c in range(RING):
        pltpu.make_async_copy(e_hbm.at[wid, c], idxr_v.at[c],
                              isem.at[c]).wait()
        pltpu.async_copy(y_hbm.at[idxr_v.at[c, 0]], stage_v.at[c],
                         gsem.at[c])

    def group(g, carry):
        for k in range(UNROLL):
            c = g * UNROLL + k
            r = k % RING
            u = k % IDXR
            un = (k + RING) % IDXR
            pltpu.make_async_copy(y_hbm.at[idxr_v.at[u, 0]], stage_v.at[r],
                                  gsem.at[r]).wait()
            pltpu.sync_copy(stage_v.at[r], acc_sh.at[idxr_v.at[u, 1]],
                            add=True)
            @pl.when(c + RING < ENCH)
            def _():
                pltpu.make_async_copy(e_hbm.at[wid, un], idxr_v.at[un],
                                      isem.at[un]).wait()
                pltpu.async_copy(y_hbm.at[idxr_v.at[un, 0]], stage_v.at[r],
                                 gsem.at[r])
            @pl.when(c + IDXR < ENCH)
            def _():
                pltpu.async_copy(e_hbm.at[wid, c + IDXR], idxr_v.at[u],
                                 isem.at[u])
        return carry

    lax.fori_loop(0, ENCH // UNROLL, group, 0)
    plsc.subcore_barrier()
    pltpu.sync_copy(acc_sh.at[pl.ds(sid * RPT, RPT)],
                    acc_out.at[cid, pl.ds(sid * RPT, RPT)])


def _t1_body(x_ref, w_ref, degp_ref, y_ref):
    deg = degp_ref[0, :, :1] + degp_ref[1, :, :1] + 1.0
    dinv = lax.rsqrt(deg)
    y_ref[...] = dinv * jnp.dot(x_ref[...], w_ref[...],
                                preferred_element_type=jnp.float32)


def _t1(x_p, W1, degp):
    return pl.pallas_call(
        _t1_body,
        grid=(NB,),
        in_specs=[
            pl.BlockSpec((BR, D), lambda i: (i, 0)),
            pl.BlockSpec((D, D), lambda i: (0, 0)),
            pl.BlockSpec((NC, BR, DEGW), lambda i: (0, i, 0)),
        ],
        out_specs=pl.BlockSpec((BR, D), lambda i: (i, 0)),
        out_shape=jax.ShapeDtypeStruct((NP, D), jnp.float32),
    )(x_p, W1, degp)


def _t2_body(accp_ref, y1_ref, degp_ref, b_ref, w_ref, y2_ref):
    deg = degp_ref[0, :, :1] + degp_ref[1, :, :1] + 1.0
    dinv = lax.rsqrt(deg)
    a = accp_ref[0] + accp_ref[1] + y1_ref[...]
    h = dinv * a + b_ref[...]
    h = jnp.where(h > 0, h, jnp.exp(jnp.minimum(h, 0.0)) - 1.0)
    y2_ref[...] = dinv * jnp.dot(h, w_ref[...],
                                 preferred_element_type=jnp.float32)


def _t2(accp, y1, degp, b1, W2):
    return pl.pallas_call(
        _t2_body,
        grid=(NB,),
        in_specs=[
            pl.BlockSpec((NC, BR, D), lambda i: (0, i, 0)),
            pl.BlockSpec((BR, D), lambda i: (i, 0)),
            pl.BlockSpec((NC, BR, DEGW), lambda i: (0, i, 0)),
            pl.BlockSpec((1, D), lambda i: (0, 0)),
            pl.BlockSpec((D, D), lambda i: (0, 0)),
        ],
        out_specs=pl.BlockSpec((BR, D), lambda i: (i, 0)),
        out_shape=jax.ShapeDtypeStruct((NP, D), jnp.float32),
    )(accp, y1, degp, b1, W2)


def _t3_body(accp_ref, y2_ref, degp_ref, b_ref, batch_ref,
             fc1w_ref, fc1b_ref, fc2w_ref, fc2b_ref, out_ref,
             pool_acc, cnt_acc):
    i = pl.program_id(0)

    @pl.when(i == 0)
    def _():
        pool_acc[...] = jnp.zeros_like(pool_acc)
        cnt_acc[...] = jnp.zeros_like(cnt_acc)

    deg = degp_ref[0, :, :1] + degp_ref[1, :, :1] + 1.0
    dinv = lax.rsqrt(deg)
    h2 = dinv * (accp_ref[0] + accp_ref[1] + y2_ref[...]) + b_ref[...]
    gids = lax.broadcasted_iota(jnp.int32, (BR, G), 1)
    onehot = (batch_ref[...] == gids).astype(jnp.float32)
    dn = (((0,), (0,)), ((), ()))
    pool_acc[...] += lax.dot_general(onehot, h2, dn,
                                     preferred_element_type=jnp.float32)
    cnt_acc[...] += lax.dot_general(onehot, jnp.ones((BR, 1), jnp.float32),
                                    dn, preferred_element_type=jnp.float32)

    @pl.when(i == NB - 1)
    def _():
        pooled = pool_acc[...] / jnp.maximum(cnt_acc[...], 1.0)
        z = jnp.dot(pooled, fc1w_ref[...],
                    preferred_element_type=jnp.float32) + fc1b_ref[...]
        z = jnp.maximum(z, 0.0)
        z = jnp.dot(z, fc2w_ref[...],
                    preferred_element_type=jnp.float32) + fc2b_ref[...]
        col = lax.broadcasted_iota(jnp.int32, (G, D), 1)
        valid = col < 10
        zm = jnp.where(valid, z, -1e30)
        m = jnp.max(zm, axis=1, keepdims=True)
        ex = jnp.where(valid, jnp.exp(zm - m), 0.0)
        s = jnp.sum(ex, axis=1, keepdims=True)
        out_ref[...] = z - m - jnp.log(s)


def _t3(accp, y2, degp, b2, batch_p, fc1w, fc1b, fc2w, fc2b):
    return pl.pallas_call(
        _t3_body,
        grid=(NB,),
        in_specs=[
            pl.BlockSpec((NC, BR, D), lambda i: (0, i, 0)),
            pl.BlockSpec((BR, D), lambda i: (i, 0)),
            pl.BlockSpec((NC, BR, DEGW), lambda i: (0, i, 0)),
            pl.BlockSpec((1, D), lambda i: (0, 0)),
            pl.BlockSpec((BR, 1), lambda i: (i, 0)),
            pl.BlockSpec((D, D), lambda i: (0, 0)),
            pl.BlockSpec((1, D), lambda i: (0, 0)),
            pl.BlockSpec((D, D), lambda i: (0, 0)),
            pl.BlockSpec((1, D), lambda i: (0, 0)),
        ],
        out_specs=pl.BlockSpec((G, D), lambda i: (0, 0)),
        out_shape=jax.ShapeDtypeStruct((G, D), jnp.float32),
        scratch_shapes=[
            pltpu.VMEM((G, D), jnp.float32),
            pltpu.VMEM((G, 1), jnp.float32),
        ],
    )(accp, y2, degp, b2, batch_p, fc1w, fc1b, fc2w, fc2b)


def kernel(x, edge_index, batch, W1, b1, W2, b2, fc1_W, fc1_b, fc2_W, fc2_b):
    f32 = jnp.float32
    x_p = jnp.zeros((NP, D), f32).at[:N].set(x)
    src = edge_index[0]
    dst = edge_index[1]
    src_p = jnp.concatenate([src, jnp.full((EP - E,), N, jnp.int32)])
    dst_p = jnp.concatenate([dst, jnp.full((EP - E,), NP - 1, jnp.int32)])
    e_r = jnp.stack([src_p.reshape(NW, ENCH, ECH),
                     dst_p.reshape(NW, ENCH, ECH)], axis=2)
    dst_rd = dst_p.reshape(NW, NCHUNK, CH)
    batch_p = jnp.full((NP, 1), G, jnp.int32).at[:N, 0].set(batch)

    ones_rows = jnp.ones((CH, DEGW), f32)
    zeros_deg = jnp.zeros((RPT, DEGW), f32)
    zeros_acc = jnp.zeros((RPT, D), f32)

    b1r = b1.reshape(1, D)
    b2r = b2.reshape(1, D)
    fc1w_p = jnp.zeros((D, D), f32).at[:, :20].set(fc1_W)
    fc1b_p = jnp.zeros((1, D), f32).at[0, :20].set(fc1_b)
    fc2w_p = jnp.zeros((D, D), f32).at[:20, :10].set(fc2_W)
    fc2b_p = jnp.zeros((1, D), f32).at[0, :10].set(fc2_b)

    degp = _sc_degree(dst_rd, ones_rows, zeros_deg)
    y1 = _t1(x_p, W1, degp)
    acc1 = _sc_edge_agg(y1, e_r, zeros_acc)
    y2 = _t2(acc1, y1, degp, b1r, W2)
    acc2 = _sc_edge_agg(y2, e_r, zeros_acc)
    out = _t3(acc2, y2, degp, b2r, batch_p, fc1w_p, fc1b_p, fc2w_p, fc2b_p)
    return out[:, :10]


# speedup vs baseline: 1.1227x; 1.1227x over previous
import functools
import jax
import jax.numpy as jnp
from jax import lax
from jax.experimental import pallas as pl
from jax.experimental.pallas import tpu as pltpu
from jax.experimental.pallas import tpu_sc as plsc

NC, NS = 2, 16
NW = NC * NS
N = 10000
NP = 10240
RPT = NP // NS
E = 320000
EP = 327680
EPT = EP // NW
CH = 128
NCHUNK = EPT // CH
ECH = 128
ENCH = EPT // ECH
D = 128
G = 128
DEGW = 128
BR = 512
NB = NP // BR

_mesh = plsc.VectorSubcoreMesh(core_axis_name="c", subcore_axis_name="s",
                               num_cores=NC, num_subcores=NS)


@functools.partial(
    pl.kernel,
    out_type=jax.ShapeDtypeStruct((NC, NP, DEGW), jnp.float32),
    mesh=_mesh,
    scratch_types=[
        pltpu.VMEM((NCHUNK, CH), jnp.int32),
        pltpu.VMEM((CH, DEGW), jnp.float32),
        pltpu.VMEM_SHARED((NP, DEGW), jnp.float32),
    ],
)
def _sc_degree(dst_hbm, ones_hbm, zeros_hbm, deg_out, idx_v, ones_v, deg_sh):
    cid = lax.axis_index("c")
    sid = lax.axis_index("s")
    wid = sid * NC + cid
    pltpu.sync_copy(zeros_hbm, deg_sh.at[pl.ds(sid * RPT, RPT)])
    pltpu.sync_copy(dst_hbm.at[wid], idx_v)
    pltpu.sync_copy(ones_hbm, ones_v)
    plsc.subcore_barrier()

    def chunk(c, carry):
        pltpu.sync_copy(ones_v, deg_sh.at[idx_v.at[c]], add=True)
        return carry

    lax.fori_loop(0, NCHUNK, chunk, 0)
    plsc.subcore_barrier()
    pltpu.sync_copy(deg_sh.at[pl.ds(sid * RPT, RPT)],
                    deg_out.at[cid, pl.ds(sid * RPT, RPT)])


RING = 2
IDXR = 8
UNROLL = 8


@functools.partial(
    pl.kernel,
    out_type=jax.ShapeDtypeStruct((NC, NP, D), jnp.float32),
    mesh=_mesh,
    scratch_types=[
        pltpu.VMEM((IDXR, 2, ECH), jnp.int32),
        pltpu.VMEM((RING, ECH, D), jnp.float32),
        pltpu.VMEM_SHARED((NP, D), jnp.float32),
        pltpu.SemaphoreType.DMA((IDXR,)),
        pltpu.SemaphoreType.DMA((RING,)),
    ],
)
def _sc_edge_agg(y_hbm, e_hbm, zeros_hbm, acc_out,
                 idxr_v, stage_v, acc_sh, isem, gsem):
    cid = lax.axis_index("c")
    sid = lax.axis_index("s")
    wid = sid * NC + cid
    pltpu.sync_copy(zeros_hbm, acc_sh.at[pl.ds(sid * RPT, RPT)])
    for u in range(IDXR):
        pltpu.async_copy(e_hbm.at[wid, u], idxr_v.at[u], isem.at[u])
    plsc.subcore_barrier()
    for c in range(RING):
        pltpu.make_async_copy(e_hbm.at[wid, c], idxr_v.at[c],
                              isem.at[c]).wait()
        pltpu.async_copy(y_hbm.at[idxr_v.at[c, 0]], stage_v.at[c],
                         gsem.at[c])

    def group(g, carry):
        for k in range(UNROLL):
            c = g * UNROLL + k
            r = k % RING
            u = k % IDXR
            un = (k + RING) % IDXR
            pltpu.make_async_copy(y_hbm.at[idxr_v.at[u, 0]], stage_v.at[r],
                                  gsem.at[r]).wait()
            pltpu.sync_copy(stage_v.at[r], acc_sh.at[idxr_v.at[u, 1]],
                            add=True)
            @pl.when(c + RING < ENCH)
            def _():
                pltpu.make_async_copy(e_hbm.at[wid, un], idxr_v.at[un],
                                      isem.at[un]).wait()
                pltpu.async_copy(y_hbm.at[idxr_v.at[un, 0]], stage_v.at[r],
                                 gsem.at[r])
            @pl.when(c + IDXR < ENCH)
            def _():
                pltpu.async_copy(e_hbm.at[wid, c + IDXR], idxr_v.at[u],
                                 isem.at[u])
        return carry

    lax.fori_loop(0, ENCH // UNROLL, group, 0)
    plsc.subcore_barrier()
    pltpu.sync_copy(acc_sh.at[pl.ds(sid * RPT, RPT)],
                    acc_out.at[cid, pl.ds(sid * RPT, RPT)])


def _t1_body(x_ref, w_ref, y_ref):
    y_ref[...] = jnp.dot(x_ref[...], w_ref[...],
                         preferred_element_type=jnp.float32)


def _t1(x_p, W1):
    return pl.pallas_call(
        _t1_body,
        grid=(NB,),
        in_specs=[
            pl.BlockSpec((BR, D), lambda i: (i, 0)),
            pl.BlockSpec((D, D), lambda i: (0, 0)),
        ],
        out_specs=pl.BlockSpec((BR, D), lambda i: (i, 0)),
        out_shape=jax.ShapeDtypeStruct((NP, D), jnp.float32),
    )(x_p, W1)


def _t1b_body(xw_ref, degp_ref, y_ref):
    deg = degp_ref[0, :, :1] + degp_ref[1, :, :1] + 1.0
    y_ref[...] = lax.rsqrt(deg) * xw_ref[...]


def _t1b(xw, degp):
    return pl.pallas_call(
        _t1b_body,
        grid=(NB,),
        in_specs=[
            pl.BlockSpec((BR, D), lambda i: (i, 0)),
            pl.BlockSpec((NC, BR, DEGW), lambda i: (0, i, 0)),
        ],
        out_specs=pl.BlockSpec((BR, D), lambda i: (i, 0)),
        out_shape=jax.ShapeDtypeStruct((NP, D), jnp.float32),
    )(xw, degp)


def _t2_body(accp_ref, y1_ref, degp_ref, b_ref, w_ref, y2_ref):
    deg = degp_ref[0, :, :1] + degp_ref[1, :, :1] + 1.0
    dinv = lax.rsqrt(deg)
    a = accp_ref[0] + accp_ref[1] + y1_ref[...]
    h = dinv * a + b_ref[...]
    h = jnp.where(h > 0, h, jnp.exp(jnp.minimum(h, 0.0)) - 1.0)
    y2_ref[...] = dinv * jnp.dot(h, w_ref[...],
                                 preferred_element_type=jnp.float32)


def _t2(accp, y1, degp, b1, W2):
    return pl.pallas_call(
        _t2_body,
        grid=(NB,),
        in_specs=[
            pl.BlockSpec((NC, BR, D), lambda i: (0, i, 0)),
            pl.BlockSpec((BR, D), lambda i: (i, 0)),
            pl.BlockSpec((NC, BR, DEGW), lambda i: (0, i, 0)),
            pl.BlockSpec((1, D), lambda i: (0, 0)),
            pl.BlockSpec((D, D), lambda i: (0, 0)),
        ],
        out_specs=pl.BlockSpec((BR, D), lambda i: (i, 0)),
        out_shape=jax.ShapeDtypeStruct((NP, D), jnp.float32),
    )(accp, y1, degp, b1, W2)


def _t3_body(accp_ref, y2_ref, degp_ref, b_ref, batch_ref,
             fc1w_ref, fc1b_ref, fc2w_ref, fc2b_ref, out_ref,
             pool_acc, cnt_acc):
    i = pl.program_id(0)

    @pl.when(i == 0)
    def _():
        pool_acc[...] = jnp.zeros_like(pool_acc)
        cnt_acc[...] = jnp.zeros_like(cnt_acc)

    deg = degp_ref[0, :, :1] + degp_ref[1, :, :1] + 1.0
    dinv = lax.rsqrt(deg)
    h2 = dinv * (accp_ref[0] + accp_ref[1] + y2_ref[...]) + b_ref[...]
    gids = lax.broadcasted_iota(jnp.int32, (BR, G), 1)
    onehot = (batch_ref[...] == gids).astype(jnp.float32)
    dn = (((0,), (0,)), ((), ()))
    pool_acc[...] += lax.dot_general(onehot, h2, dn,
                                     preferred_element_type=jnp.float32)
    cnt_acc[...] += lax.dot_general(onehot, jnp.ones((BR, 1), jnp.float32),
                                    dn, preferred_element_type=jnp.float32)

    @pl.when(i == NB - 1)
    def _():
        pooled = pool_acc[...] / jnp.maximum(cnt_acc[...], 1.0)
        z = jnp.dot(pooled, fc1w_ref[...],
                    preferred_element_type=jnp.float32) + fc1b_ref[...]
        z = jnp.maximum(z, 0.0)
        z = jnp.dot(z, fc2w_ref[...],
                    preferred_element_type=jnp.float32) + fc2b_ref[...]
        col = lax.broadcasted_iota(jnp.int32, (G, D), 1)
        valid = col < 10
        zm = jnp.where(valid, z, -1e30)
        m = jnp.max(zm, axis=1, keepdims=True)
        ex = jnp.where(valid, jnp.exp(zm - m), 0.0)
        s = jnp.sum(ex, axis=1, keepdims=True)
        out_ref[...] = z - m - jnp.log(s)


def _t3(accp, y2, degp, b2, batch_p, fc1w, fc1b, fc2w, fc2b):
    return pl.pallas_call(
        _t3_body,
        grid=(NB,),
        in_specs=[
            pl.BlockSpec((NC, BR, D), lambda i: (0, i, 0)),
            pl.BlockSpec((BR, D), lambda i: (i, 0)),
            pl.BlockSpec((NC, BR, DEGW), lambda i: (0, i, 0)),
            pl.BlockSpec((1, D), lambda i: (0, 0)),
            pl.BlockSpec((BR, 1), lambda i: (i, 0)),
            pl.BlockSpec((D, D), lambda i: (0, 0)),
            pl.BlockSpec((1, D), lambda i: (0, 0)),
            pl.BlockSpec((D, D), lambda i: (0, 0)),
            pl.BlockSpec((1, D), lambda i: (0, 0)),
        ],
        out_specs=pl.BlockSpec((G, D), lambda i: (0, 0)),
        out_shape=jax.ShapeDtypeStruct((G, D), jnp.float32),
        scratch_shapes=[
            pltpu.VMEM((G, D), jnp.float32),
            pltpu.VMEM((G, 1), jnp.float32),
        ],
    )(accp, y2, degp, b2, batch_p, fc1w, fc1b, fc2w, fc2b)


def kernel(x, edge_index, batch, W1, b1, W2, b2, fc1_W, fc1_b, fc2_W, fc2_b):
    f32 = jnp.float32
    x_p = jnp.zeros((NP, D), f32).at[:N].set(x)
    src = edge_index[0]
    dst = edge_index[1]
    src_p = jnp.concatenate([src, jnp.full((EP - E,), N, jnp.int32)])
    dst_p = jnp.concatenate([dst, jnp.full((EP - E,), NP - 1, jnp.int32)])
    e_r = jnp.stack([src_p.reshape(NW, ENCH, ECH),
                     dst_p.reshape(NW, ENCH, ECH)], axis=2)
    dst_rd = dst_p.reshape(NW, NCHUNK, CH)
    batch_p = jnp.full((NP, 1), G, jnp.int32).at[:N, 0].set(batch)

    ones_rows = jnp.ones((CH, DEGW), f32)
    zeros_deg = jnp.zeros((RPT, DEGW), f32)
    zeros_acc = jnp.zeros((RPT, D), f32)

    b1r = b1.reshape(1, D)
    b2r = b2.reshape(1, D)
    fc1w_p = jnp.zeros((D, D), f32).at[:, :20].set(fc1_W)
    fc1b_p = jnp.zeros((1, D), f32).at[0, :20].set(fc1_b)
    fc2w_p = jnp.zeros((D, D), f32).at[:20, :10].set(fc2_W)
    fc2b_p = jnp.zeros((1, D), f32).at[0, :10].set(fc2_b)

    xw = _t1(x_p, W1)
    degp = _sc_degree(dst_rd, ones_rows, zeros_deg)
    y1 = _t1b(xw, degp)
    acc1 = _sc_edge_agg(y1, e_r, zeros_acc)
    y2 = _t2(acc1, y1, degp, b1r, W2)
    acc2 = _sc_edge_agg(y2, e_r, zeros_acc)
    out = _t3(acc2, y2, degp, b2r, batch_p, fc1w_p, fc1b_p, fc2w_p, fc2b_p)
    return out[:, :10]
